# baseline (speedup 1.0000x reference)
import functools

import jax
import jax.numpy as jnp
from jax import lax
from jax.experimental import pallas as pl
from jax.experimental.pallas import tpu as pltpu
from jax.experimental.pallas import tpu_sc as plsc

NUM_ROWS = 1_000_000
DIM = 64
NA, NT = 4096, 200
BATCH = NA * NT
NC, NS = 2, 16
NW = NC * NS
BPW = BATCH // NW
CHUNK = 128
NCH = BPW // CHUNK
NBUF = 8

_mesh = plsc.VectorSubcoreMesh(core_axis_name="c", subcore_axis_name="s")


@functools.partial(
    pl.kernel,
    mesh=_mesh,
    out_type=jax.ShapeDtypeStruct((BATCH, DIM), jnp.float32),
    compiler_params=pltpu.CompilerParams(use_tc_tiling_on_sc=False),
    scratch_types=[
        pltpu.VMEM((BPW,), jnp.int32),
        *[pltpu.VMEM((CHUNK, DIM), jnp.float32) for _ in range(NBUF)],
        pltpu.SemaphoreType.DMA,
        *[pltpu.SemaphoreType.DMA for _ in range(NBUF)],
        *[pltpu.SemaphoreType.DMA for _ in range(NBUF)],
    ],
)
def _gather(x_hbm, w_hbm, out_hbm, idx_v, *bufs):
    rows = bufs[0:NBUF]
    semi = bufs[NBUF]
    sgs = bufs[NBUF + 1:2 * NBUF + 1]
    sos = bufs[2 * NBUF + 1:]

    wid = lax.axis_index("s") * NC + lax.axis_index("c")
    base = wid * BPW

    pltpu.make_async_copy(x_hbm.at[pl.ds(base, BPW)], idx_v, semi).start()
    pltpu.make_async_copy(x_hbm.at[pl.ds(base, BPW)], idx_v, semi).wait()

    def start_gather(j, p):
        pltpu.make_async_copy(
            w_hbm.at[idx_v.at[pl.ds(j * CHUNK, CHUNK)]], rows[p], sgs[p]
        ).start()

    for p in range(NBUF - 2):
        start_gather(p, p)

    def body(i, carry):
        for p in range(NBUF):
            j = NBUF * i + p

            pltpu.make_async_copy(
                w_hbm.at[idx_v.at[pl.ds(j * CHUNK, CHUNK)]], rows[p], sgs[p]
            ).wait()
            pltpu.make_async_copy(
                rows[p],
                out_hbm.at[pl.ds(base + j * CHUNK, CHUNK), :],
                sos[p],
            ).start()

            pw = (p + NBUF - 2) % NBUF

            @pl.when(j >= 2)
            def _():
                pltpu.make_async_copy(
                    rows[pw], out_hbm.at[pl.ds(base, CHUNK), :], sos[pw]
                ).wait()

            @pl.when(j + NBUF - 2 < NCH)
            def _():
                start_gather(j + NBUF - 2, pw)

        return carry

    lax.fori_loop(0, NCH // NBUF, body, 0)

    for j in (NCH - 2, NCH - 1):
        p = j % NBUF
        pltpu.make_async_copy(
            rows[p], out_hbm.at[pl.ds(base, CHUNK), :], sos[p]
        ).wait()


def kernel(x, weight):
    x1 = x.reshape(BATCH).astype(jnp.int32)
    out = _gather(x1, weight)
    return out.reshape(NA, NT, DIM)
